# Initial kernel scaffold; baseline (speedup 1.0000x reference)
import jax
import jax.numpy as jnp
from jax.experimental import pallas as pl


def kernel(node_features, edge_features, positions, msg_w1, msg_b1, msg_w2, msg_b2, upd_w1, upd_b1, upd_w2, upd_b2, ln_g, ln_b, edge_index):
    raise NotImplementedError("write your pallas kernel here")


# speedup vs baseline: 1.9574x; 1.9574x over previous
import functools

import jax
import jax.numpy as jnp
from jax import lax
from jax.experimental import pallas as pl
from jax.experimental.pallas import tpu as pltpu
from jax.experimental.pallas import tpu_sc as plsc

N = 10000
E = 320000
D = 128
H = 128
TW = 144

NC = 2
NS = 16
NW = NC * NS
EPW = E // NW
CH = 80
NCHUNK = EPW // CH
RPT = N // NS
RQ = 125
NQ = RPT // RQ

BN = 512
BE = 512


def _prep_body(nf_ref, pos_ref, w_ref, out_ref):
    p = jnp.dot(nf_ref[...], w_ref[...], preferred_element_type=jnp.float32)
    pad = jnp.zeros((p.shape[0], TW - D - 3), jnp.float32)
    out_ref[...] = jnp.concatenate([p, pos_ref[...], pad], axis=-1)


def _tc_prep(nf, pos, w1_nf):
    return pl.pallas_call(
        _prep_body,
        grid=(pl.cdiv(N, BN),),
        in_specs=[
            pl.BlockSpec((BN, D), lambda i: (i, 0)),
            pl.BlockSpec((BN, 3), lambda i: (i, 0)),
            pl.BlockSpec((D, H), lambda i: (0, 0)),
        ],
        out_specs=pl.BlockSpec((BN, TW), lambda i: (i, 0)),
        out_shape=jax.ShapeDtypeStruct((N, TW), jnp.float32),
    )(nf, pos, w1_nf)


_sc_mesh = plsc.VectorSubcoreMesh(core_axis_name="c", subcore_axis_name="s")
_sc_params = pltpu.CompilerParams(use_tc_tiling_on_sc=False)


@functools.partial(
    pl.kernel,
    mesh=_sc_mesh,
    compiler_params=_sc_params,
    out_type=[
        jax.ShapeDtypeStruct((E, TW), jnp.float32),
        jax.ShapeDtypeStruct((E, TW), jnp.float32),
    ],
    scratch_types=[
        pltpu.VMEM((CH,), jnp.int32),
        pltpu.VMEM((CH,), jnp.int32),
        pltpu.VMEM((CH, TW), jnp.float32),
        pltpu.VMEM((CH, TW), jnp.float32),
        pltpu.SemaphoreType.DMA,
        pltpu.SemaphoreType.DMA,
    ],
)
def _sc_gather(t_hbm, row_hbm, col_hbm,
               gcol_hbm, grow_hbm,
               idxr_v, idxc_v, gcol_v, grow_v, sem1, sem2):
    cid = lax.axis_index("c")
    sid = lax.axis_index("s")
    wid = cid * NS + sid
    base0 = wid * EPW

    def body(i, carry):
        b = base0 + i * CH
        pltpu.sync_copy(row_hbm.at[pl.ds(b, CH)], idxr_v)
        pltpu.sync_copy(col_hbm.at[pl.ds(b, CH)], idxc_v)
        cp1 = pltpu.async_copy(t_hbm.at[idxc_v], gcol_v, sem1)
        cp2 = pltpu.async_copy(t_hbm.at[idxr_v], grow_v, sem2)
        cp1.wait()
        cp2.wait()
        pltpu.sync_copy(gcol_v, gcol_hbm.at[pl.ds(b, CH)])
        pltpu.sync_copy(grow_v, grow_hbm.at[pl.ds(b, CH)])
        return carry

    lax.fori_loop(0, NCHUNK, body, 0)


def _silu(x):
    return x * jax.nn.sigmoid(x)


def _msg_body(gcol_ref, grow_ref, ef_ref,
              wtail_ref, w1l_ref, b1_ref, w2_ref, b2_ref,
              mr_ref, mc_ref):
    gcol = gcol_ref[...]
    grow = grow_ref[...]
    dpos = grow[:, D:D + 3] - gcol[:, D:D + 3]
    rdsq = jnp.sum(dpos * dpos, axis=-1, keepdims=True)
    et = (jnp.dot(ef_ref[...], wtail_ref[...],
                  preferred_element_type=jnp.float32)
          + rdsq * w1l_ref[...] + b1_ref[...])
    hr = _silu(gcol[:, :D] + et)
    mr_ref[...] = _silu(jnp.dot(hr, w2_ref[...],
                                preferred_element_type=jnp.float32)
                        + b2_ref[...])
    hc = _silu(grow[:, :D] + et)
    mc_ref[...] = _silu(jnp.dot(hc, w2_ref[...],
                                preferred_element_type=jnp.float32)
                        + b2_ref[...])


def _tc_msg(gcol, grow, ef, wtail, w1l, b1, w2, b2):
    full = lambda r, c: pl.BlockSpec((r, c), lambda i: (0, 0))
    return pl.pallas_call(
        _msg_body,
        grid=(E // BE,),
        in_specs=[
            pl.BlockSpec((BE, TW), lambda i: (i, 0)),
            pl.BlockSpec((BE, TW), lambda i: (i, 0)),
            pl.BlockSpec((BE, 16), lambda i: (i, 0)),
            full(16, H), full(1, H), full(1, H), full(H, H), full(1, H),
        ],
        out_specs=[
            pl.BlockSpec((BE, H), lambda i: (i, 0)),
            pl.BlockSpec((BE, H), lambda i: (i, 0)),
        ],
        out_shape=[
            jax.ShapeDtypeStruct((E, H), jnp.float32),
            jax.ShapeDtypeStruct((E, H), jnp.float32),
        ],
    )(gcol, grow, ef, wtail, w1l, b1, w2, b2)


@functools.partial(
    pl.kernel,
    mesh=_sc_mesh,
    compiler_params=_sc_params,
    out_type=[
        jax.ShapeDtypeStruct((N, H), jnp.float32),
        jax.ShapeDtypeStruct((N, H), jnp.float32),
    ],
    scratch_types=[
        pltpu.VMEM_SHARED((N, H), jnp.float32),
        pltpu.VMEM((RQ, H), jnp.float32),
        pltpu.VMEM((CH,), jnp.int32),
        pltpu.VMEM((CH, H), jnp.float32),
    ],
)
def _sc_scatter(mr_hbm, mc_hbm, row_hbm, col_hbm,
                part0_hbm, part1_hbm,
                acc_sh, zbuf_v, idx_v, dat_v):
    cid = lax.axis_index("c")
    sid = lax.axis_index("s")
    base0 = (cid * NS + sid) * EPW

    def zb(r, carry):
        for k in range(H // 16):
            zbuf_v[r, pl.ds(k * 16, 16)] = jnp.zeros((16,), jnp.float32)
        return carry

    lax.fori_loop(0, RQ, zb, 0)
    for q in range(NQ):
        pltpu.sync_copy(zbuf_v, acc_sh.at[pl.ds(sid * RPT + q * RQ, RQ), :])
    plsc.subcore_barrier()

    def body(i, carry):
        b = base0 + i * CH
        pltpu.sync_copy(row_hbm.at[pl.ds(b, CH)], idx_v)
        pltpu.sync_copy(mr_hbm.at[pl.ds(b, CH)], dat_v)
        pltpu.sync_copy(dat_v, acc_sh.at[idx_v], add=True)
        pltpu.sync_copy(col_hbm.at[pl.ds(b, CH)], idx_v)
        pltpu.sync_copy(mc_hbm.at[pl.ds(b, CH)], dat_v)
        pltpu.sync_copy(dat_v, acc_sh.at[idx_v], add=True)
        return carry

    lax.fori_loop(0, NCHUNK, body, 0)
    plsc.subcore_barrier()

    @pl.when(cid == 0)
    def _():
        for q in range(NQ):
            r0 = sid * RPT + q * RQ
            pltpu.sync_copy(acc_sh.at[pl.ds(r0, RQ), :],
                            part0_hbm.at[pl.ds(r0, RQ), :])

    @pl.when(cid == 1)
    def _():
        for q in range(NQ):
            r0 = sid * RPT + q * RQ
            pltpu.sync_copy(acc_sh.at[pl.ds(r0, RQ), :],
                            part1_hbm.at[pl.ds(r0, RQ), :])


def _upd_body(nf_ref, p0_ref, p1_ref,
              u1a_ref, u1b_ref, ub1_ref, uw2_ref, ub2_ref, g_ref, bb_ref,
              out_ref):
    nf = nf_ref[...]
    agg = p0_ref[...] + p1_ref[...]
    h = _silu(jnp.dot(nf, u1a_ref[...], preferred_element_type=jnp.float32)
              + jnp.dot(agg, u1b_ref[...], preferred_element_type=jnp.float32)
              + ub1_ref[...])
    upd = jnp.dot(h, uw2_ref[...],
                  preferred_element_type=jnp.float32) + ub2_ref[...]
    x = nf + upd
    mu = jnp.mean(x, axis=-1, keepdims=True)
    var = jnp.mean((x - mu) ** 2, axis=-1, keepdims=True)
    out_ref[...] = (x - mu) * lax.rsqrt(var + 1e-5) * g_ref[...] + bb_ref[...]


def _tc_update(nf, p0, p1, u1a, u1b, ub1, uw2, ub2, g, bb):
    full = lambda r, c: pl.BlockSpec((r, c), lambda i: (0, 0))
    return pl.pallas_call(
        _upd_body,
        grid=(pl.cdiv(N, BN),),
        in_specs=[
            pl.BlockSpec((BN, D), lambda i: (i, 0)),
            pl.BlockSpec((BN, H), lambda i: (i, 0)),
            pl.BlockSpec((BN, H), lambda i: (i, 0)),
            full(D, H), full(H, H), full(1, H), full(H, D), full(1, D),
            full(1, D), full(1, D),
        ],
        out_specs=pl.BlockSpec((BN, D), lambda i: (i, 0)),
        out_shape=jax.ShapeDtypeStruct((N, D), jnp.float32),
    )(nf, p0, p1, u1a, u1b, ub1, uw2, ub2, g, bb)


def kernel(node_features, edge_features, positions,
           msg_w1, msg_b1, msg_w2, msg_b2,
           upd_w1, upd_b1, upd_w2, upd_b2,
           ln_g, ln_b, edge_index):
    row = edge_index[0].astype(jnp.int32)
    col = edge_index[1].astype(jnp.int32)

    w1_nf = msg_w1[:D]
    wtail = jnp.zeros((16, H), jnp.float32).at[3:16].set(msg_w1[D:D + 13])
    w1l = msg_w1[141:142]
    b1 = msg_b1[None]
    b2 = msg_b2[None]
    u1a = upd_w1[:D]
    u1b = upd_w1[D:]
    ub1 = upd_b1[None]
    ub2 = upd_b2[None]
    g = ln_g[None]
    bb = ln_b[None]

    t = _tc_prep(node_features, positions, w1_nf)
    gcol, grow = _sc_gather(t, row, col)
    mr, mc = _tc_msg(gcol, grow, edge_features, wtail, w1l, b1, msg_w2, b2)
    p0, p1 = _sc_scatter(mr, mc, row, col)
    return _tc_update(node_features, p0, p1,
                      u1a, u1b, ub1, upd_w2, ub2, g, bb)


# speedup vs baseline: 1.9681x; 1.0055x over previous
import functools

import jax
import jax.numpy as jnp
from jax import lax
from jax.experimental import pallas as pl
from jax.experimental.pallas import tpu as pltpu
from jax.experimental.pallas import tpu_sc as plsc

N = 10000
E = 320000
D = 128
H = 128

NC = 2
NS = 16
NW = NC * NS
EPW = E // NW
CH = 80
NCHUNK = EPW // CH
RPT = N // NS
RQ = 125
NQ = RPT // RQ

BN = 512
BE = 512


def _prep_body(nf_ref, w_ref, out_ref):
    out_ref[...] = jnp.dot(nf_ref[...], w_ref[...],
                           preferred_element_type=jnp.float32
                           ).astype(jnp.bfloat16)


def _tc_prep(nf, w1_nf):
    return pl.pallas_call(
        _prep_body,
        grid=(pl.cdiv(N, BN),),
        in_specs=[
            pl.BlockSpec((BN, D), lambda i: (i, 0)),
            pl.BlockSpec((D, H), lambda i: (0, 0)),
        ],
        out_specs=pl.BlockSpec((BN, H), lambda i: (i, 0)),
        out_shape=jax.ShapeDtypeStruct((N, H), jnp.bfloat16),
    )(nf, w1_nf)


_sc_mesh = plsc.VectorSubcoreMesh(core_axis_name="c", subcore_axis_name="s")
_sc_params = pltpu.CompilerParams(use_tc_tiling_on_sc=False)


@functools.partial(
    pl.kernel,
    mesh=_sc_mesh,
    compiler_params=_sc_params,
    out_type=[
        jax.ShapeDtypeStruct((E, H), jnp.bfloat16),
        jax.ShapeDtypeStruct((E, H), jnp.bfloat16),
        jax.ShapeDtypeStruct((E,), jnp.float32),
    ],
    scratch_types=[
        pltpu.VMEM((CH,), jnp.int32),
        pltpu.VMEM((CH,), jnp.int32),
        pltpu.VMEM((CH, H), jnp.bfloat16),
        pltpu.VMEM((CH, H), jnp.bfloat16),
        pltpu.VMEM((CH, 16), jnp.float32),
        pltpu.VMEM((CH, 16), jnp.float32),
        pltpu.VMEM((CH,), jnp.float32),
        pltpu.SemaphoreType.DMA,
        pltpu.SemaphoreType.DMA,
        pltpu.SemaphoreType.DMA,
    ],
)
def _sc_gather(p_hbm, pos_hbm, row_hbm, col_hbm,
               gcol_hbm, grow_hbm, rdsq_hbm,
               idxr_v, idxc_v, gcol_v, grow_v, posr_v, posc_v, rdsq_v,
               sem1, sem2, sem3):
    cid = lax.axis_index("c")
    sid = lax.axis_index("s")
    wid = cid * NS + sid
    base0 = wid * EPW

    def body(i, carry):
        b = base0 + i * CH
        pltpu.sync_copy(row_hbm.at[pl.ds(b, CH)], idxr_v)
        pltpu.sync_copy(col_hbm.at[pl.ds(b, CH)], idxc_v)
        cp1 = pltpu.async_copy(p_hbm.at[idxc_v], gcol_v, sem1)
        cp2 = pltpu.async_copy(p_hbm.at[idxr_v], grow_v, sem2)
        cp3 = pltpu.async_copy(pos_hbm.at[idxr_v], posr_v, sem3)
        cp4 = pltpu.async_copy(pos_hbm.at[idxc_v], posc_v, sem3)
        cp1.wait()
        cp2.wait()
        pltpu.sync_copy(gcol_v, gcol_hbm.at[pl.ds(b, CH)])
        pltpu.sync_copy(grow_v, grow_hbm.at[pl.ds(b, CH)])
        cp3.wait()
        cp4.wait()

        lanes = lax.broadcasted_iota(jnp.int32, (16,), 0)

        def grp(gi, c2):
            def rowf(j, vec):
                r = gi * 16 + j
                d = posr_v[r, pl.ds(0, 16)] - posc_v[r, pl.ds(0, 16)]
                sq = d * d
                s = sq[0] + sq[1] + sq[2]
                return jnp.where(lanes == j, s, vec)

            vec = lax.fori_loop(0, 16, rowf, jnp.zeros((16,), jnp.float32))
            rdsq_v[pl.ds(gi * 16, 16)] = vec
            return c2

        lax.fori_loop(0, CH // 16, grp, 0)
        pltpu.sync_copy(rdsq_v, rdsq_hbm.at[pl.ds(b, CH)])
        return carry

    lax.fori_loop(0, NCHUNK, body, 0)


def _silu(x):
    return x * jax.nn.sigmoid(x)


def _msg_body(gcol_ref, grow_ref, ef_ref,
              wtail_ref, b1_ref, w2_ref, b2_ref,
              mr_ref, mc_ref):
    et = (jnp.dot(ef_ref[...], wtail_ref[...],
                  preferred_element_type=jnp.float32)
          + b1_ref[...])
    hr = _silu(gcol_ref[...].astype(jnp.float32) + et)
    mr_ref[...] = _silu(jnp.dot(hr, w2_ref[...],
                                preferred_element_type=jnp.float32)
                        + b2_ref[...])
    hc = _silu(grow_ref[...].astype(jnp.float32) + et)
    mc_ref[...] = _silu(jnp.dot(hc, w2_ref[...],
                                preferred_element_type=jnp.float32)
                        + b2_ref[...])


def _tc_msg(gcol, grow, ef2, wtail, b1, w2, b2):
    full = lambda r, c: pl.BlockSpec((r, c), lambda i: (0, 0))
    return pl.pallas_call(
        _msg_body,
        grid=(E // BE,),
        in_specs=[
            pl.BlockSpec((BE, H), lambda i: (i, 0)),
            pl.BlockSpec((BE, H), lambda i: (i, 0)),
            pl.BlockSpec((BE, 16), lambda i: (i, 0)),
            full(16, H), full(1, H), full(H, H), full(1, H),
        ],
        out_specs=[
            pl.BlockSpec((BE, H), lambda i: (i, 0)),
            pl.BlockSpec((BE, H), lambda i: (i, 0)),
        ],
        out_shape=[
            jax.ShapeDtypeStruct((E, H), jnp.float32),
            jax.ShapeDtypeStruct((E, H), jnp.float32),
        ],
    )(gcol, grow, ef2, wtail, b1, w2, b2)


@functools.partial(
    pl.kernel,
    mesh=_sc_mesh,
    compiler_params=_sc_params,
    out_type=[
        jax.ShapeDtypeStruct((2 * N, H), jnp.float32),
    ],
    scratch_types=[
        pltpu.VMEM_SHARED((N, H), jnp.float32),
        pltpu.VMEM((RQ, H), jnp.float32),
        pltpu.VMEM((CH,), jnp.int32),
        pltpu.VMEM((CH, H), jnp.float32),
    ],
)
def _sc_scatter(mr_hbm, mc_hbm, row_hbm, col_hbm,
                part_hbm,
                acc_sh, zbuf_v, idx_v, dat_v):
    cid = lax.axis_index("c")
    sid = lax.axis_index("s")
    base0 = (cid * NS + sid) * EPW

    def zb(r, carry):
        for k in range(H // 16):
            zbuf_v[r, pl.ds(k * 16, 16)] = jnp.zeros((16,), jnp.float32)
        return carry

    lax.fori_loop(0, RQ, zb, 0)
    for q in range(NQ):
        pltpu.sync_copy(zbuf_v, acc_sh.at[pl.ds(sid * RPT + q * RQ, RQ), :])
    plsc.subcore_barrier()

    def body(i, carry):
        b = base0 + i * CH
        pltpu.sync_copy(row_hbm.at[pl.ds(b, CH)], idx_v)
        pltpu.sync_copy(mr_hbm.at[pl.ds(b, CH)], dat_v)
        pltpu.sync_copy(dat_v, acc_sh.at[idx_v], add=True)
        pltpu.sync_copy(col_hbm.at[pl.ds(b, CH)], idx_v)
        pltpu.sync_copy(mc_hbm.at[pl.ds(b, CH)], dat_v)
        pltpu.sync_copy(dat_v, acc_sh.at[idx_v], add=True)
        return carry

    lax.fori_loop(0, NCHUNK, body, 0)
    plsc.subcore_barrier()

    for q in range(NQ):
        r0 = sid * RPT + q * RQ
        pltpu.sync_copy(acc_sh.at[pl.ds(r0, RQ), :],
                        part_hbm.at[pl.ds(cid * N + r0, RQ), :])


def _upd_body(nf_ref, p0_ref, p1_ref,
              u1a_ref, u1b_ref, ub1_ref, uw2_ref, ub2_ref, g_ref, bb_ref,
              out_ref):
    nf = nf_ref[...]
    agg = p0_ref[...] + p1_ref[...]
    h = _silu(jnp.dot(nf, u1a_ref[...], preferred_element_type=jnp.float32)
              + jnp.dot(agg, u1b_ref[...], preferred_element_type=jnp.float32)
              + ub1_ref[...])
    upd = jnp.dot(h, uw2_ref[...],
                  preferred_element_type=jnp.float32) + ub2_ref[...]
    x = nf + upd
    mu = jnp.mean(x, axis=-1, keepdims=True)
    var = jnp.mean((x - mu) ** 2, axis=-1, keepdims=True)
    out_ref[...] = (x - mu) * lax.rsqrt(var + 1e-5) * g_ref[...] + bb_ref[...]


def _tc_update(nf, p0, p1, u1a, u1b, ub1, uw2, ub2, g, bb):
    full = lambda r, c: pl.BlockSpec((r, c), lambda i: (0, 0))
    return pl.pallas_call(
        _upd_body,
        grid=(pl.cdiv(N, BN),),
        in_specs=[
            pl.BlockSpec((BN, D), lambda i: (i, 0)),
            pl.BlockSpec((BN, H), lambda i: (i, 0)),
            pl.BlockSpec((BN, H), lambda i: (i, 0)),
            full(D, H), full(H, H), full(1, H), full(H, D), full(1, D),
            full(1, D), full(1, D),
        ],
        out_specs=pl.BlockSpec((BN, D), lambda i: (i, 0)),
        out_shape=jax.ShapeDtypeStruct((N, D), jnp.float32),
    )(nf, p0, p1, u1a, u1b, ub1, uw2, ub2, g, bb)


def kernel(node_features, edge_features, positions,
           msg_w1, msg_b1, msg_w2, msg_b2,
           upd_w1, upd_b1, upd_w2, upd_b2,
           ln_g, ln_b, edge_index):
    row = edge_index[0].astype(jnp.int32)
    col = edge_index[1].astype(jnp.int32)

    w1_nf = msg_w1[:D]
    wtail = (jnp.zeros((16, H), jnp.float32)
             .at[3:16].set(msg_w1[D:D + 13])
             .at[0].set(msg_w1[141]))
    b1 = msg_b1[None]
    b2 = msg_b2[None]
    u1a = upd_w1[:D]
    u1b = upd_w1[D:]
    ub1 = upd_b1[None]
    ub2 = upd_b2[None]
    g = ln_g[None]
    bb = ln_b[None]

    pos16 = jnp.pad(positions, ((0, 0), (0, 13)))

    p = _tc_prep(node_features, w1_nf)
    gcol, grow, rdsq = _sc_gather(p, pos16, row, col)
    ef2 = jnp.concatenate([rdsq[:, None], edge_features[:, 1:]], axis=1)
    mr, mc = _tc_msg(gcol, grow, ef2, wtail, b1, msg_w2, b2)
    part, = _sc_scatter(mr, mc, row, col)
    return _tc_update(node_features, part[:N], part[N:],
                      u1a, u1b, ub1, upd_w2, ub2, g, bb)


# speedup vs baseline: 2.5775x; 1.3096x over previous
import functools

import jax
import jax.numpy as jnp
from jax import lax
from jax.experimental import pallas as pl
from jax.experimental.pallas import tpu as pltpu
from jax.experimental.pallas import tpu_sc as plsc

N = 10000
E = 320000
D = 128
H = 128

NC = 2
NS = 16
NW = NC * NS
EPW = E // NW
CH = 80
NCHUNK = EPW // CH
RPT = N // NS
RQ = 125
NQ = RPT // RQ

BN = 512
BE = 512


def _prep_body(nf_ref, w_ref, out_ref):
    out_ref[...] = jnp.dot(nf_ref[...], w_ref[...],
                           preferred_element_type=jnp.float32)


def _tc_prep(nf, w1_nf):
    return pl.pallas_call(
        _prep_body,
        grid=(pl.cdiv(N, BN),),
        in_specs=[
            pl.BlockSpec((BN, D), lambda i: (i, 0)),
            pl.BlockSpec((D, H), lambda i: (0, 0)),
        ],
        out_specs=pl.BlockSpec((BN, H), lambda i: (i, 0)),
        out_shape=jax.ShapeDtypeStruct((N, H), jnp.float32),
    )(nf, w1_nf)


_sc_mesh = plsc.VectorSubcoreMesh(core_axis_name="c", subcore_axis_name="s")
_sc_params = pltpu.CompilerParams(use_tc_tiling_on_sc=False)


@functools.partial(
    pl.kernel,
    mesh=_sc_mesh,
    compiler_params=_sc_params,
    out_type=[
        jax.ShapeDtypeStruct((E, H), jnp.float32),
        jax.ShapeDtypeStruct((E, H), jnp.float32),
        jax.ShapeDtypeStruct((E,), jnp.float32),
    ],
    scratch_types=[
        pltpu.VMEM((CH,), jnp.int32),
        pltpu.VMEM((CH,), jnp.int32),
        pltpu.VMEM((CH, H), jnp.float32),
        pltpu.VMEM((CH, H), jnp.float32),
        pltpu.VMEM((CH, 16), jnp.float32),
        pltpu.VMEM((CH, 16), jnp.float32),
        pltpu.VMEM((CH,), jnp.float32),
        pltpu.SemaphoreType.DMA,
        pltpu.SemaphoreType.DMA,
        pltpu.SemaphoreType.DMA,
    ],
)
def _sc_gather(p_hbm, pos_hbm, row_hbm, col_hbm,
               gcol_hbm, grow_hbm, rdsq_hbm,
               idxr_v, idxc_v, gcol_v, grow_v, posr_v, posc_v, rdsq_v,
               sem1, sem2, sem3):
    cid = lax.axis_index("c")
    sid = lax.axis_index("s")
    wid = cid * NS + sid
    base0 = wid * EPW

    def body(i, carry):
        b = base0 + i * CH
        pltpu.sync_copy(row_hbm.at[pl.ds(b, CH)], idxr_v)
        pltpu.sync_copy(col_hbm.at[pl.ds(b, CH)], idxc_v)
        cp1 = pltpu.async_copy(p_hbm.at[idxc_v], gcol_v, sem1)
        cp2 = pltpu.async_copy(p_hbm.at[idxr_v], grow_v, sem2)
        cp3 = pltpu.async_copy(pos_hbm.at[idxr_v], posr_v, sem3)
        cp4 = pltpu.async_copy(pos_hbm.at[idxc_v], posc_v, sem3)
        cp1.wait()
        cp2.wait()
        pltpu.sync_copy(gcol_v, gcol_hbm.at[pl.ds(b, CH)])
        pltpu.sync_copy(grow_v, grow_hbm.at[pl.ds(b, CH)])
        cp3.wait()
        cp4.wait()

        lanes = lax.broadcasted_iota(jnp.int32, (16,), 0)

        def grp(gi, c2):
            def rowf(j, vec):
                r = gi * 16 + j
                d = posr_v[r, pl.ds(0, 16)] - posc_v[r, pl.ds(0, 16)]
                sq = d * d
                s = sq[0] + sq[1] + sq[2]
                return jnp.where(lanes == j, s, vec)

            vec = lax.fori_loop(0, 16, rowf, jnp.zeros((16,), jnp.float32))
            rdsq_v[pl.ds(gi * 16, 16)] = vec
            return c2

        lax.fori_loop(0, CH // 16, grp, 0)
        pltpu.sync_copy(rdsq_v, rdsq_hbm.at[pl.ds(b, CH)])
        return carry

    lax.fori_loop(0, NCHUNK, body, 0)


def _silu(x):
    return x * jax.nn.sigmoid(x)


def _msg_body(gcol_ref, grow_ref, ef_ref,
              wtail_ref, b1_ref, w2_ref, b2_ref,
              mr_ref, mc_ref):
    et = (jnp.dot(ef_ref[...], wtail_ref[...],
                  preferred_element_type=jnp.float32)
          + b1_ref[...])
    hr = _silu(gcol_ref[...] + et)
    mr_ref[...] = _silu(jnp.dot(hr, w2_ref[...],
                                preferred_element_type=jnp.float32)
                        + b2_ref[...])
    hc = _silu(grow_ref[...] + et)
    mc_ref[...] = _silu(jnp.dot(hc, w2_ref[...],
                                preferred_element_type=jnp.float32)
                        + b2_ref[...])


def _tc_msg(gcol, grow, ef2, wtail, b1, w2, b2):
    full = lambda r, c: pl.BlockSpec((r, c), lambda i: (0, 0))
    return pl.pallas_call(
        _msg_body,
        grid=(E // BE,),
        in_specs=[
            pl.BlockSpec((BE, H), lambda i: (i, 0)),
            pl.BlockSpec((BE, H), lambda i: (i, 0)),
            pl.BlockSpec((BE, 16), lambda i: (i, 0)),
            full(16, H), full(1, H), full(H, H), full(1, H),
        ],
        out_specs=[
            pl.BlockSpec((BE, H), lambda i: (i, 0)),
            pl.BlockSpec((BE, H), lambda i: (i, 0)),
        ],
        out_shape=[
            jax.ShapeDtypeStruct((E, H), jnp.float32),
            jax.ShapeDtypeStruct((E, H), jnp.float32),
        ],
    )(gcol, grow, ef2, wtail, b1, w2, b2)


@functools.partial(
    pl.kernel,
    mesh=_sc_mesh,
    compiler_params=_sc_params,
    out_type=[
        jax.ShapeDtypeStruct((2 * N, H), jnp.float32),
    ],
    scratch_types=[
        pltpu.VMEM_SHARED((N, H), jnp.float32),
        pltpu.VMEM((RQ, H), jnp.float32),
        pltpu.VMEM((CH,), jnp.int32),
        pltpu.VMEM((CH, H), jnp.float32),
    ],
)
def _sc_scatter(mr_hbm, mc_hbm, row_hbm, col_hbm,
                part_hbm,
                acc_sh, zbuf_v, idx_v, dat_v):
    cid = lax.axis_index("c")
    sid = lax.axis_index("s")
    base0 = (cid * NS + sid) * EPW

    def zb(r, carry):
        for k in range(H // 16):
            zbuf_v[r, pl.ds(k * 16, 16)] = jnp.zeros((16,), jnp.float32)
        return carry

    lax.fori_loop(0, RQ, zb, 0)
    for q in range(NQ):
        pltpu.sync_copy(zbuf_v, acc_sh.at[pl.ds(sid * RPT + q * RQ, RQ), :])
    plsc.subcore_barrier()

    def body(i, carry):
        b = base0 + i * CH
        pltpu.sync_copy(row_hbm.at[pl.ds(b, CH)], idx_v)
        pltpu.sync_copy(mr_hbm.at[pl.ds(b, CH)], dat_v)
        pltpu.sync_copy(dat_v, acc_sh.at[idx_v], add=True)
        pltpu.sync_copy(col_hbm.at[pl.ds(b, CH)], idx_v)
        pltpu.sync_copy(mc_hbm.at[pl.ds(b, CH)], dat_v)
        pltpu.sync_copy(dat_v, acc_sh.at[idx_v], add=True)
        return carry

    lax.fori_loop(0, NCHUNK, body, 0)
    plsc.subcore_barrier()

    for q in range(NQ):
        r0 = sid * RPT + q * RQ
        pltpu.sync_copy(acc_sh.at[pl.ds(r0, RQ), :],
                        part_hbm.at[pl.ds(cid * N + r0, RQ), :])


def _upd_body(nf_ref, p0_ref, p1_ref,
              u1a_ref, u1b_ref, ub1_ref, uw2_ref, ub2_ref, g_ref, bb_ref,
              out_ref):
    nf = nf_ref[...]
    agg = p0_ref[...] + p1_ref[...]
    h = _silu(jnp.dot(nf, u1a_ref[...], preferred_element_type=jnp.float32)
              + jnp.dot(agg, u1b_ref[...], preferred_element_type=jnp.float32)
              + ub1_ref[...])
    upd = jnp.dot(h, uw2_ref[...],
                  preferred_element_type=jnp.float32) + ub2_ref[...]
    x = nf + upd
    mu = jnp.mean(x, axis=-1, keepdims=True)
    var = jnp.mean((x - mu) ** 2, axis=-1, keepdims=True)
    out_ref[...] = (x - mu) * lax.rsqrt(var + 1e-5) * g_ref[...] + bb_ref[...]


def _tc_update(nf, p0, p1, u1a, u1b, ub1, uw2, ub2, g, bb):
    full = lambda r, c: pl.BlockSpec((r, c), lambda i: (0, 0))
    return pl.pallas_call(
        _upd_body,
        grid=(pl.cdiv(N, BN),),
        in_specs=[
            pl.BlockSpec((BN, D), lambda i: (i, 0)),
            pl.BlockSpec((BN, H), lambda i: (i, 0)),
            pl.BlockSpec((BN, H), lambda i: (i, 0)),
            full(D, H), full(H, H), full(1, H), full(H, D), full(1, D),
            full(1, D), full(1, D),
        ],
        out_specs=pl.BlockSpec((BN, D), lambda i: (i, 0)),
        out_shape=jax.ShapeDtypeStruct((N, D), jnp.float32),
    )(nf, p0, p1, u1a, u1b, ub1, uw2, ub2, g, bb)


def kernel(node_features, edge_features, positions,
           msg_w1, msg_b1, msg_w2, msg_b2,
           upd_w1, upd_b1, upd_w2, upd_b2,
           ln_g, ln_b, edge_index):
    row = edge_index[0].astype(jnp.int32)
    col = edge_index[1].astype(jnp.int32)

    w1_nf = msg_w1[:D]
    wtail = (jnp.zeros((16, H), jnp.float32)
             .at[3:16].set(msg_w1[D:D + 13])
             .at[0].set(msg_w1[141]))
    b1 = msg_b1[None]
    b2 = msg_b2[None]
    u1a = upd_w1[:D]
    u1b = upd_w1[D:]
    ub1 = upd_b1[None]
    ub2 = upd_b2[None]
    g = ln_g[None]
    bb = ln_b[None]

    pos16 = jnp.pad(positions, ((0, 0), (0, 13)))

    p = _tc_prep(node_features, w1_nf)
    gcol, grow, rdsq = _sc_gather(p, pos16, row, col)
    ef2 = jnp.concatenate([rdsq[:, None], edge_features[:, 1:]], axis=1)
    mr, mc = _tc_msg(gcol, grow, ef2, wtail, b1, msg_w2, b2)
    part, = _sc_scatter(mr, mc, row, col)
    return _tc_update(node_features, part[:N], part[N:],
                      u1a, u1b, ub1, upd_w2, ub2, g, bb)


# speedup vs baseline: 2.9258x; 1.1351x over previous
import functools

import jax
import jax.numpy as jnp
from jax import lax
from jax.experimental import pallas as pl
from jax.experimental.pallas import tpu as pltpu
from jax.experimental.pallas import tpu_sc as plsc

N = 10000
E = 320000
D = 128
H = 128

NC = 2
NS = 16
NW = NC * NS
EPW = E // NW
CH = 80
NCHUNK = EPW // CH
RPT = N // NS
RQ = 125
NQ = RPT // RQ

BN = 512
BE = 1280


def _prep_body(nf_ref, w_ref, out_ref):
    out_ref[...] = jnp.dot(nf_ref[...], w_ref[...],
                           preferred_element_type=jnp.float32)


def _tc_prep(nf, w1_nf):
    return pl.pallas_call(
        _prep_body,
        grid=(pl.cdiv(N, BN),),
        in_specs=[
            pl.BlockSpec((BN, D), lambda i: (i, 0)),
            pl.BlockSpec((D, H), lambda i: (0, 0)),
        ],
        out_specs=pl.BlockSpec((BN, H), lambda i: (i, 0)),
        out_shape=jax.ShapeDtypeStruct((N, H), jnp.float32),
    )(nf, w1_nf)


_sc_mesh = plsc.VectorSubcoreMesh(core_axis_name="c", subcore_axis_name="s")
_sc_params = pltpu.CompilerParams(use_tc_tiling_on_sc=False)


@functools.partial(
    pl.kernel,
    mesh=_sc_mesh,
    compiler_params=_sc_params,
    out_type=[
        jax.ShapeDtypeStruct((E, H), jnp.float32),
        jax.ShapeDtypeStruct((E, H), jnp.float32),
        jax.ShapeDtypeStruct((E,), jnp.float32),
    ],
    scratch_types=[
        pltpu.VMEM((CH,), jnp.int32),
        pltpu.VMEM((CH,), jnp.int32),
        pltpu.VMEM((CH, H), jnp.float32),
        pltpu.VMEM((CH, H), jnp.float32),
        pltpu.VMEM((CH, 16), jnp.float32),
        pltpu.VMEM((CH, 16), jnp.float32),
        pltpu.VMEM((CH,), jnp.float32),
        pltpu.SemaphoreType.DMA,
        pltpu.SemaphoreType.DMA,
        pltpu.SemaphoreType.DMA,
    ],
)
def _sc_gather(p_hbm, pos_hbm, row_hbm, col_hbm,
               gcol_hbm, grow_hbm, rdsq_hbm,
               idxr_v, idxc_v, gcol_v, grow_v, posr_v, posc_v, rdsq_v,
               sem1, sem2, sem3):
    cid = lax.axis_index("c")
    sid = lax.axis_index("s")
    wid = cid * NS + sid
    base0 = wid * EPW

    def body(i, carry):
        b = base0 + i * CH
        pltpu.sync_copy(row_hbm.at[pl.ds(b, CH)], idxr_v)
        pltpu.sync_copy(col_hbm.at[pl.ds(b, CH)], idxc_v)
        cp1 = pltpu.async_copy(p_hbm.at[idxc_v], gcol_v, sem1)
        cp2 = pltpu.async_copy(p_hbm.at[idxr_v], grow_v, sem2)
        cp3 = pltpu.async_copy(pos_hbm.at[idxr_v], posr_v, sem3)
        cp4 = pltpu.async_copy(pos_hbm.at[idxc_v], posc_v, sem3)
        cp1.wait()
        cp2.wait()
        pltpu.sync_copy(gcol_v, gcol_hbm.at[pl.ds(b, CH)])
        pltpu.sync_copy(grow_v, grow_hbm.at[pl.ds(b, CH)])
        cp3.wait()
        cp4.wait()

        lanes = lax.broadcasted_iota(jnp.int32, (16,), 0)

        def grp(gi, c2):
            def rowf(j, vec):
                r = gi * 16 + j
                d = posr_v[r, pl.ds(0, 16)] - posc_v[r, pl.ds(0, 16)]
                sq = d * d
                s = sq[0] + sq[1] + sq[2]
                return jnp.where(lanes == j, s, vec)

            vec = lax.fori_loop(0, 16, rowf, jnp.zeros((16,), jnp.float32))
            rdsq_v[pl.ds(gi * 16, 16)] = vec
            return c2

        lax.fori_loop(0, CH // 16, grp, 0)
        pltpu.sync_copy(rdsq_v, rdsq_hbm.at[pl.ds(b, CH)])
        return carry

    lax.fori_loop(0, NCHUNK, body, 0)


def _silu(x):
    return x * jax.nn.sigmoid(x)


def _msg_body(gcol_ref, grow_ref, ef_ref,
              wtail_ref, b1_ref, w2_ref, b2_ref,
              mr_ref, mc_ref):
    et = (jnp.dot(ef_ref[...], wtail_ref[...],
                  preferred_element_type=jnp.float32)
          + b1_ref[...])
    w2b = w2_ref[...].astype(jnp.bfloat16)
    hr = _silu(gcol_ref[...] + et).astype(jnp.bfloat16)
    mr_ref[...] = _silu(jnp.dot(hr, w2b,
                                preferred_element_type=jnp.float32)
                        + b2_ref[...])
    hc = _silu(grow_ref[...] + et).astype(jnp.bfloat16)
    mc_ref[...] = _silu(jnp.dot(hc, w2b,
                                preferred_element_type=jnp.float32)
                        + b2_ref[...])


def _tc_msg(gcol, grow, ef2, wtail, b1, w2, b2):
    full = lambda r, c: pl.BlockSpec((r, c), lambda i: (0, 0))
    return pl.pallas_call(
        _msg_body,
        grid=(E // BE,),
        in_specs=[
            pl.BlockSpec((BE, H), lambda i: (i, 0)),
            pl.BlockSpec((BE, H), lambda i: (i, 0)),
            pl.BlockSpec((BE, 16), lambda i: (i, 0)),
            full(16, H), full(1, H), full(H, H), full(1, H),
        ],
        out_specs=[
            pl.BlockSpec((BE, H), lambda i: (i, 0)),
            pl.BlockSpec((BE, H), lambda i: (i, 0)),
        ],
        out_shape=[
            jax.ShapeDtypeStruct((E, H), jnp.float32),
            jax.ShapeDtypeStruct((E, H), jnp.float32),
        ],
    )(gcol, grow, ef2, wtail, b1, w2, b2)


@functools.partial(
    pl.kernel,
    mesh=_sc_mesh,
    compiler_params=_sc_params,
    out_type=[
        jax.ShapeDtypeStruct((2 * N, H), jnp.float32),
    ],
    scratch_types=[
        pltpu.VMEM_SHARED((N, H), jnp.float32),
        pltpu.VMEM((RQ, H), jnp.float32),
        pltpu.VMEM((CH,), jnp.int32),
        pltpu.VMEM((CH, H), jnp.float32),
    ],
)
def _sc_scatter(mr_hbm, mc_hbm, row_hbm, col_hbm,
                part_hbm,
                acc_sh, zbuf_v, idx_v, dat_v):
    cid = lax.axis_index("c")
    sid = lax.axis_index("s")
    base0 = (cid * NS + sid) * EPW

    def zb(r, carry):
        for k in range(H // 16):
            zbuf_v[r, pl.ds(k * 16, 16)] = jnp.zeros((16,), jnp.float32)
        return carry

    lax.fori_loop(0, RQ, zb, 0)
    for q in range(NQ):
        pltpu.sync_copy(zbuf_v, acc_sh.at[pl.ds(sid * RPT + q * RQ, RQ), :])
    plsc.subcore_barrier()

    def body(i, carry):
        b = base0 + i * CH
        pltpu.sync_copy(row_hbm.at[pl.ds(b, CH)], idx_v)
        pltpu.sync_copy(mr_hbm.at[pl.ds(b, CH)], dat_v)
        pltpu.sync_copy(dat_v, acc_sh.at[idx_v], add=True)
        pltpu.sync_copy(col_hbm.at[pl.ds(b, CH)], idx_v)
        pltpu.sync_copy(mc_hbm.at[pl.ds(b, CH)], dat_v)
        pltpu.sync_copy(dat_v, acc_sh.at[idx_v], add=True)
        return carry

    lax.fori_loop(0, NCHUNK, body, 0)
    plsc.subcore_barrier()

    for q in range(NQ):
        r0 = sid * RPT + q * RQ
        pltpu.sync_copy(acc_sh.at[pl.ds(r0, RQ), :],
                        part_hbm.at[pl.ds(cid * N + r0, RQ), :])


def _upd_body(nf_ref, p0_ref, p1_ref,
              u1a_ref, u1b_ref, ub1_ref, uw2_ref, ub2_ref, g_ref, bb_ref,
              out_ref):
    nf = nf_ref[...]
    agg = p0_ref[...] + p1_ref[...]
    h = _silu(jnp.dot(nf, u1a_ref[...], preferred_element_type=jnp.float32)
              + jnp.dot(agg, u1b_ref[...], preferred_element_type=jnp.float32)
              + ub1_ref[...])
    upd = jnp.dot(h, uw2_ref[...],
                  preferred_element_type=jnp.float32) + ub2_ref[...]
    x = nf + upd
    mu = jnp.mean(x, axis=-1, keepdims=True)
    var = jnp.mean((x - mu) ** 2, axis=-1, keepdims=True)
    out_ref[...] = (x - mu) * lax.rsqrt(var + 1e-5) * g_ref[...] + bb_ref[...]


def _tc_update(nf, p0, p1, u1a, u1b, ub1, uw2, ub2, g, bb):
    full = lambda r, c: pl.BlockSpec((r, c), lambda i: (0, 0))
    return pl.pallas_call(
        _upd_body,
        grid=(pl.cdiv(N, BN),),
        in_specs=[
            pl.BlockSpec((BN, D), lambda i: (i, 0)),
            pl.BlockSpec((BN, H), lambda i: (i, 0)),
            pl.BlockSpec((BN, H), lambda i: (i, 0)),
            full(D, H), full(H, H), full(1, H), full(H, D), full(1, D),
            full(1, D), full(1, D),
        ],
        out_specs=pl.BlockSpec((BN, D), lambda i: (i, 0)),
        out_shape=jax.ShapeDtypeStruct((N, D), jnp.float32),
    )(nf, p0, p1, u1a, u1b, ub1, uw2, ub2, g, bb)


def kernel(node_features, edge_features, positions,
           msg_w1, msg_b1, msg_w2, msg_b2,
           upd_w1, upd_b1, upd_w2, upd_b2,
           ln_g, ln_b, edge_index):
    row = edge_index[0].astype(jnp.int32)
    col = edge_index[1].astype(jnp.int32)

    w1_nf = msg_w1[:D]
    wtail = (jnp.zeros((16, H), jnp.float32)
             .at[3:16].set(msg_w1[D:D + 13])
             .at[0].set(msg_w1[141]))
    b1 = msg_b1[None]
    b2 = msg_b2[None]
    u1a = upd_w1[:D]
    u1b = upd_w1[D:]
    ub1 = upd_b1[None]
    ub2 = upd_b2[None]
    g = ln_g[None]
    bb = ln_b[None]

    pos16 = jnp.pad(positions, ((0, 0), (0, 13)))

    p = _tc_prep(node_features, w1_nf)
    gcol, grow, rdsq = _sc_gather(p, pos16, row, col)
    ef2 = jnp.concatenate([rdsq[:, None], edge_features[:, 1:]], axis=1)
    mr, mc = _tc_msg(gcol, grow, ef2, wtail, b1, msg_w2, b2)
    part, = _sc_scatter(mr, mc, row, col)
    return _tc_update(node_features, part[:N], part[N:],
                      u1a, u1b, ub1, upd_w2, ub2, g, bb)


# speedup vs baseline: 3.6507x; 1.2477x over previous
import functools

import jax
import jax.numpy as jnp
from jax import lax
from jax.experimental import pallas as pl
from jax.experimental.pallas import tpu as pltpu
from jax.experimental.pallas import tpu_sc as plsc

N = 10000
E = 320000
D = 128
H = 128

NC = 2
NS = 16
NW = NC * NS
EPW = E // NW
CH = 80
NCHUNK = EPW // CH
RPT = N // NS
RQ = 125
NQ = RPT // RQ
ZR = 25
NZQ = RPT // ZR

BN = 512
BE = 1280


def _prep_body(nf_ref, w_ref, out_ref):
    out_ref[...] = jnp.dot(nf_ref[...], w_ref[...],
                           preferred_element_type=jnp.float32)


def _tc_prep(nf, w1_nf):
    return pl.pallas_call(
        _prep_body,
        grid=(pl.cdiv(N, BN),),
        in_specs=[
            pl.BlockSpec((BN, D), lambda i: (i, 0)),
            pl.BlockSpec((D, H), lambda i: (0, 0)),
        ],
        out_specs=pl.BlockSpec((BN, H), lambda i: (i, 0)),
        out_shape=jax.ShapeDtypeStruct((N, H), jnp.float32),
    )(nf, w1_nf)


_sc_mesh = plsc.VectorSubcoreMesh(core_axis_name="c", subcore_axis_name="s")
_sc_params = pltpu.CompilerParams(use_tc_tiling_on_sc=False)


@functools.partial(
    pl.kernel,
    mesh=_sc_mesh,
    compiler_params=_sc_params,
    out_type=[
        jax.ShapeDtypeStruct((E, H), jnp.float32),
        jax.ShapeDtypeStruct((E, H), jnp.float32),
        jax.ShapeDtypeStruct((E, 16), jnp.float32),
    ],
    scratch_types=[
        pltpu.VMEM((CH,), jnp.int32), pltpu.VMEM((CH,), jnp.int32),
        pltpu.VMEM((CH,), jnp.int32), pltpu.VMEM((CH,), jnp.int32),
        pltpu.VMEM((CH, H), jnp.float32), pltpu.VMEM((CH, H), jnp.float32),
        pltpu.VMEM((CH, H), jnp.float32), pltpu.VMEM((CH, H), jnp.float32),
        pltpu.VMEM((CH, 16), jnp.float32), pltpu.VMEM((CH, 16), jnp.float32),
        pltpu.VMEM((CH, 16), jnp.float32), pltpu.VMEM((CH, 16), jnp.float32),
        pltpu.VMEM((CH, 16), jnp.float32), pltpu.VMEM((CH, 16), jnp.float32),
        pltpu.SemaphoreType.DMA, pltpu.SemaphoreType.DMA,
        pltpu.SemaphoreType.DMA, pltpu.SemaphoreType.DMA,
    ],
)
def _sc_gather(p_hbm, pos_hbm, ef_hbm, row_hbm, col_hbm,
               gcol_hbm, grow_hbm, ef2_hbm,
               idxr0, idxc0, idxr1, idxc1,
               gc0, gr0, gc1, gr1,
               pr0, pc0, pr1, pc1, ef0, ef1,
               semg0, semg1, semw0, semw1):
    cid = lax.axis_index("c")
    sid = lax.axis_index("s")
    base0 = (cid * NS + sid) * EPW
    lanes = lax.broadcasted_iota(jnp.int32, (16,), 0)

    bufA = (idxr0, idxc0, gc0, gr0, pr0, pc0, ef0, semg0, semw0)
    bufB = (idxr1, idxc1, gc1, gr1, pr1, pc1, ef1, semg1, semw1)

    def start(i, buf):
        idxr, idxc, gc, gr, pr, pc, ef, semg, _ = buf
        b = base0 + i * CH
        pltpu.sync_copy(row_hbm.at[pl.ds(b, CH)], idxr)
        pltpu.sync_copy(col_hbm.at[pl.ds(b, CH)], idxc)
        pltpu.sync_copy(ef_hbm.at[pl.ds(b, CH), :], ef)
        pltpu.async_copy(p_hbm.at[idxc], gc, semg)
        pltpu.async_copy(p_hbm.at[idxr], gr, semg)
        pltpu.async_copy(pos_hbm.at[idxr], pr, semg)
        pltpu.async_copy(pos_hbm.at[idxc], pc, semg)

    def wait_g(buf):
        idxr, idxc, gc, gr, pr, pc, ef, semg, _ = buf
        pltpu.make_async_copy(p_hbm.at[idxc], gc, semg).wait()
        pltpu.make_async_copy(p_hbm.at[idxr], gr, semg).wait()
        pltpu.make_async_copy(pos_hbm.at[idxr], pr, semg).wait()
        pltpu.make_async_copy(pos_hbm.at[idxc], pc, semg).wait()

    def compute(buf):
        _, _, _, _, pr, pc, ef, _, _ = buf

        def rowf(r, c2):
            d = pr[r, pl.ds(0, 16)] - pc[r, pl.ds(0, 16)]
            sq = d * d
            s = sq[0] + sq[1] + sq[2]
            e = ef[r, pl.ds(0, 16)]
            ef[r, pl.ds(0, 16)] = jnp.where(lanes == 0, s, e)
            return c2

        lax.fori_loop(0, CH, rowf, 0)

    def start_w(i, buf):
        _, _, gc, gr, _, _, ef, _, semw = buf
        b = base0 + i * CH
        pltpu.async_copy(gc, gcol_hbm.at[pl.ds(b, CH)], semw)
        pltpu.async_copy(gr, grow_hbm.at[pl.ds(b, CH)], semw)
        pltpu.async_copy(ef, ef2_hbm.at[pl.ds(b, CH), :], semw)

    def drain_w(buf):
        _, _, gc, gr, _, _, ef, _, semw = buf
        pltpu.make_async_copy(gc, gcol_hbm.at[pl.ds(0, CH)], semw).wait()
        pltpu.make_async_copy(gr, grow_hbm.at[pl.ds(0, CH)], semw).wait()
        pltpu.make_async_copy(ef, ef2_hbm.at[pl.ds(0, CH), :], semw).wait()

    start(0, bufA)

    def body2(g, carry):
        @pl.when(g > 0)
        def _():
            drain_w(bufB)

        start(2 * g + 1, bufB)
        wait_g(bufA)
        compute(bufA)
        start_w(2 * g, bufA)
        wait_g(bufB)
        compute(bufB)
        start_w(2 * g + 1, bufB)
        drain_w(bufA)
        start(2 * g + 2, bufA)
        return carry

    lax.fori_loop(0, NCHUNK // 2, body2, 0)
    drain_w(bufB)
    wait_g(bufA)
    compute(bufA)
    start_w(NCHUNK - 1, bufA)
    drain_w(bufA)


def _silu(x):
    return x * jax.nn.sigmoid(x)


def _msg_body(gcol_ref, grow_ref, ef_ref,
              wtail_ref, b1_ref, w2_ref, b2_ref,
              mr_ref, mc_ref):
    et = (jnp.dot(ef_ref[...], wtail_ref[...],
                  preferred_element_type=jnp.float32)
          + b1_ref[...])
    w2b = w2_ref[...].astype(jnp.bfloat16)
    hr = _silu(gcol_ref[...] + et).astype(jnp.bfloat16)
    mr_ref[...] = _silu(jnp.dot(hr, w2b,
                                preferred_element_type=jnp.float32)
                        + b2_ref[...])
    hc = _silu(grow_ref[...] + et).astype(jnp.bfloat16)
    mc_ref[...] = _silu(jnp.dot(hc, w2b,
                                preferred_element_type=jnp.float32)
                        + b2_ref[...])


def _tc_msg(gcol, grow, ef2, wtail, b1, w2, b2):
    full = lambda r, c: pl.BlockSpec((r, c), lambda i: (0, 0))
    return pl.pallas_call(
        _msg_body,
        grid=(E // BE,),
        in_specs=[
            pl.BlockSpec((BE, H), lambda i: (i, 0)),
            pl.BlockSpec((BE, H), lambda i: (i, 0)),
            pl.BlockSpec((BE, 16), lambda i: (i, 0)),
            full(16, H), full(1, H), full(H, H), full(1, H),
        ],
        out_specs=[
            pl.BlockSpec((BE, H), lambda i: (i, 0)),
            pl.BlockSpec((BE, H), lambda i: (i, 0)),
        ],
        out_shape=[
            jax.ShapeDtypeStruct((E, H), jnp.float32),
            jax.ShapeDtypeStruct((E, H), jnp.float32),
        ],
    )(gcol, grow, ef2, wtail, b1, w2, b2)


@functools.partial(
    pl.kernel,
    mesh=_sc_mesh,
    compiler_params=_sc_params,
    out_type=[
        jax.ShapeDtypeStruct((2 * N, H), jnp.float32),
    ],
    scratch_types=[
        pltpu.VMEM_SHARED((N, H), jnp.float32),
        pltpu.VMEM((ZR, H), jnp.float32),
        pltpu.VMEM((CH,), jnp.int32), pltpu.VMEM((CH,), jnp.int32),
        pltpu.VMEM((CH,), jnp.int32), pltpu.VMEM((CH,), jnp.int32),
        pltpu.VMEM((CH, H), jnp.float32), pltpu.VMEM((CH, H), jnp.float32),
        pltpu.VMEM((CH, H), jnp.float32), pltpu.VMEM((CH, H), jnp.float32),
        pltpu.SemaphoreType.DMA, pltpu.SemaphoreType.DMA,
    ],
)
def _sc_scatter(mr_hbm, mc_hbm, row_hbm, col_hbm,
                part_hbm,
                acc_sh, zbuf_v,
                ir0, ic0, ir1, ic1, mr0, mc0, mr1, mc1, sem0, sem1):
    cid = lax.axis_index("c")
    sid = lax.axis_index("s")
    base0 = (cid * NS + sid) * EPW

    def zb(r, carry):
        for k in range(H // 16):
            zbuf_v[r, pl.ds(k * 16, 16)] = jnp.zeros((16,), jnp.float32)
        return carry

    lax.fori_loop(0, ZR, zb, 0)

    def zc(q, carry):
        pltpu.sync_copy(zbuf_v, acc_sh.at[pl.ds(sid * RPT + q * ZR, ZR), :])
        return carry

    lax.fori_loop(0, NZQ, zc, 0)
    plsc.subcore_barrier()

    bufA = (ir0, ic0, mr0, mc0, sem0)
    bufB = (ir1, ic1, mr1, mc1, sem1)

    def start(i, buf):
        ir, ic, mrv, mcv, sem = buf
        b = base0 + i * CH
        pltpu.async_copy(row_hbm.at[pl.ds(b, CH)], ir, sem)
        pltpu.async_copy(col_hbm.at[pl.ds(b, CH)], ic, sem)
        pltpu.async_copy(mr_hbm.at[pl.ds(b, CH)], mrv, sem)
        pltpu.async_copy(mc_hbm.at[pl.ds(b, CH)], mcv, sem)

    def finish(buf):
        ir, ic, mrv, mcv, sem = buf
        pltpu.make_async_copy(row_hbm.at[pl.ds(0, CH)], ir, sem).wait()
        pltpu.make_async_copy(col_hbm.at[pl.ds(0, CH)], ic, sem).wait()
        pltpu.make_async_copy(mr_hbm.at[pl.ds(0, CH)], mrv, sem).wait()
        pltpu.make_async_copy(mc_hbm.at[pl.ds(0, CH)], mcv, sem).wait()
        pltpu.sync_copy(mrv, acc_sh.at[ir], add=True)
        pltpu.sync_copy(mcv, acc_sh.at[ic], add=True)

    start(0, bufA)

    def body2(g, carry):
        start(2 * g + 1, bufB)
        finish(bufA)
        start(2 * g + 2, bufA)
        finish(bufB)
        return carry

    lax.fori_loop(0, NCHUNK // 2, body2, 0)
    finish(bufA)
    plsc.subcore_barrier()

    for q in range(NQ):
        r0 = sid * RPT + q * RQ
        pltpu.sync_copy(acc_sh.at[pl.ds(r0, RQ), :],
                        part_hbm.at[pl.ds(cid * N + r0, RQ), :])


def _upd_body(nf_ref, p0_ref, p1_ref,
              u1a_ref, u1b_ref, ub1_ref, uw2_ref, ub2_ref, g_ref, bb_ref,
              out_ref):
    nf = nf_ref[...]
    agg = p0_ref[...] + p1_ref[...]
    h = _silu(jnp.dot(nf, u1a_ref[...], preferred_element_type=jnp.float32)
              + jnp.dot(agg, u1b_ref[...], preferred_element_type=jnp.float32)
              + ub1_ref[...])
    upd = jnp.dot(h, uw2_ref[...],
                  preferred_element_type=jnp.float32) + ub2_ref[...]
    x = nf + upd
    mu = jnp.mean(x, axis=-1, keepdims=True)
    var = jnp.mean((x - mu) ** 2, axis=-1, keepdims=True)
    out_ref[...] = (x - mu) * lax.rsqrt(var + 1e-5) * g_ref[...] + bb_ref[...]


def _tc_update(nf, p0, p1, u1a, u1b, ub1, uw2, ub2, g, bb):
    full = lambda r, c: pl.BlockSpec((r, c), lambda i: (0, 0))
    return pl.pallas_call(
        _upd_body,
        grid=(pl.cdiv(N, BN),),
        in_specs=[
            pl.BlockSpec((BN, D), lambda i: (i, 0)),
            pl.BlockSpec((BN, H), lambda i: (i, 0)),
            pl.BlockSpec((BN, H), lambda i: (i, 0)),
            full(D, H), full(H, H), full(1, H), full(H, D), full(1, D),
            full(1, D), full(1, D),
        ],
        out_specs=pl.BlockSpec((BN, D), lambda i: (i, 0)),
        out_shape=jax.ShapeDtypeStruct((N, D), jnp.float32),
    )(nf, p0, p1, u1a, u1b, ub1, uw2, ub2, g, bb)


def kernel(node_features, edge_features, positions,
           msg_w1, msg_b1, msg_w2, msg_b2,
           upd_w1, upd_b1, upd_w2, upd_b2,
           ln_g, ln_b, edge_index):
    row = edge_index[0].astype(jnp.int32)
    col = edge_index[1].astype(jnp.int32)

    w1_nf = msg_w1[:D]
    wtail = (jnp.zeros((16, H), jnp.float32)
             .at[3:16].set(msg_w1[D:D + 13])
             .at[0].set(msg_w1[141]))
    b1 = msg_b1[None]
    b2 = msg_b2[None]
    u1a = upd_w1[:D]
    u1b = upd_w1[D:]
    ub1 = upd_b1[None]
    ub2 = upd_b2[None]
    g = ln_g[None]
    bb = ln_b[None]

    pos16 = jnp.pad(positions, ((0, 0), (0, 13)))

    p = _tc_prep(node_features, w1_nf)
    gcol, grow, ef2 = _sc_gather(p, pos16, edge_features, row, col)
    mr, mc = _tc_msg(gcol, grow, ef2, wtail, b1, msg_w2, b2)
    part, = _sc_scatter(mr, mc, row, col)
    return _tc_update(node_features, part[:N], part[N:],
                      u1a, u1b, ub1, upd_w2, ub2, g, bb)


# speedup vs baseline: 4.0225x; 1.1018x over previous
import functools

import jax
import jax.numpy as jnp
from jax import lax
from jax.experimental import pallas as pl
from jax.experimental.pallas import tpu as pltpu
from jax.experimental.pallas import tpu_sc as plsc

N = 10000
E = 320000
D = 128
H = 128

NC = 2
NS = 16
NW = NC * NS
EPW = E // NW
CH = 80
NCHUNK = EPW // CH
RPT = N // NS
RQ = 125
NQ = RPT // RQ
ZR = 25
NZQ = RPT // ZR

BN = 512
BE = 1280


def _prep_body(nf_ref, w_ref, out_ref):
    out_ref[...] = jnp.dot(nf_ref[...], w_ref[...],
                           preferred_element_type=jnp.float32)


def _tc_prep(nf, w1_nf):
    return pl.pallas_call(
        _prep_body,
        grid=(pl.cdiv(N, BN),),
        in_specs=[
            pl.BlockSpec((BN, D), lambda i: (i, 0)),
            pl.BlockSpec((D, H), lambda i: (0, 0)),
        ],
        out_specs=pl.BlockSpec((BN, H), lambda i: (i, 0)),
        out_shape=jax.ShapeDtypeStruct((N, H), jnp.float32),
    )(nf, w1_nf)


_sc_mesh = plsc.VectorSubcoreMesh(core_axis_name="c", subcore_axis_name="s")
_sc_params = pltpu.CompilerParams(use_tc_tiling_on_sc=False)


@functools.partial(
    pl.kernel,
    mesh=_sc_mesh,
    compiler_params=_sc_params,
    out_type=[
        jax.ShapeDtypeStruct((E, H), jnp.float32),
        jax.ShapeDtypeStruct((E, H), jnp.float32),
        jax.ShapeDtypeStruct((E,), jnp.float32),
    ],
    scratch_types=[
        pltpu.VMEM((CH,), jnp.int32), pltpu.VMEM((CH,), jnp.int32),
        pltpu.VMEM((CH,), jnp.int32), pltpu.VMEM((CH,), jnp.int32),
        pltpu.VMEM((CH, H), jnp.float32), pltpu.VMEM((CH, H), jnp.float32),
        pltpu.VMEM((CH, H), jnp.float32), pltpu.VMEM((CH, H), jnp.float32),
        pltpu.VMEM((CH, 16), jnp.float32), pltpu.VMEM((CH, 16), jnp.float32),
        pltpu.VMEM((CH, 16), jnp.float32), pltpu.VMEM((CH, 16), jnp.float32),
        pltpu.VMEM((CH,), jnp.float32), pltpu.VMEM((CH,), jnp.float32),
        pltpu.SemaphoreType.DMA, pltpu.SemaphoreType.DMA,
        pltpu.SemaphoreType.DMA, pltpu.SemaphoreType.DMA,
    ],
)
def _sc_gather(p_hbm, pos_hbm, row_hbm, col_hbm,
               gcol_hbm, grow_hbm, rdsq_hbm,
               idxr0, idxc0, idxr1, idxc1,
               gc0, gr0, gc1, gr1,
               pr0, pc0, pr1, pc1, ef0, ef1,
               semg0, semg1, semw0, semw1):
    cid = lax.axis_index("c")
    sid = lax.axis_index("s")
    base0 = (cid * NS + sid) * EPW
    lanes = lax.broadcasted_iota(jnp.int32, (16,), 0)

    bufA = (idxr0, idxc0, gc0, gr0, pr0, pc0, ef0, semg0, semw0)
    bufB = (idxr1, idxc1, gc1, gr1, pr1, pc1, ef1, semg1, semw1)

    def start(i, buf):
        idxr, idxc, gc, gr, pr, pc, ef, semg, _ = buf
        b = base0 + i * CH
        pltpu.sync_copy(row_hbm.at[pl.ds(b, CH)], idxr)
        pltpu.sync_copy(col_hbm.at[pl.ds(b, CH)], idxc)
        pltpu.async_copy(p_hbm.at[idxc], gc, semg)
        pltpu.async_copy(p_hbm.at[idxr], gr, semg)
        pltpu.async_copy(pos_hbm.at[idxr], pr, semg)
        pltpu.async_copy(pos_hbm.at[idxc], pc, semg)

    def wait_g(buf):
        idxr, idxc, gc, gr, pr, pc, ef, semg, _ = buf
        pltpu.make_async_copy(p_hbm.at[idxc], gc, semg).wait()
        pltpu.make_async_copy(p_hbm.at[idxr], gr, semg).wait()
        pltpu.make_async_copy(pos_hbm.at[idxr], pr, semg).wait()
        pltpu.make_async_copy(pos_hbm.at[idxc], pc, semg).wait()

    def compute(buf):
        _, _, _, _, pr, pc, ef, _, _ = buf

        def grp(gi, c2):
            def rowf(j, vec):
                r = gi * 16 + j
                d = pr[r, pl.ds(0, 16)] - pc[r, pl.ds(0, 16)]
                sq = d * d
                s = sq[0] + sq[1] + sq[2]
                return jnp.where(lanes == j, s, vec)

            vec = lax.fori_loop(0, 16, rowf, jnp.zeros((16,), jnp.float32))
            ef[pl.ds(gi * 16, 16)] = vec
            return c2

        lax.fori_loop(0, CH // 16, grp, 0)

    def start_w(i, buf):
        _, _, gc, gr, _, _, ef, _, semw = buf
        b = base0 + i * CH
        pltpu.async_copy(gc, gcol_hbm.at[pl.ds(b, CH)], semw)
        pltpu.async_copy(gr, grow_hbm.at[pl.ds(b, CH)], semw)
        pltpu.async_copy(ef, rdsq_hbm.at[pl.ds(b, CH)], semw)

    def drain_w(buf):
        _, _, gc, gr, _, _, ef, _, semw = buf
        pltpu.make_async_copy(gc, gcol_hbm.at[pl.ds(0, CH)], semw).wait()
        pltpu.make_async_copy(gr, grow_hbm.at[pl.ds(0, CH)], semw).wait()
        pltpu.make_async_copy(ef, rdsq_hbm.at[pl.ds(0, CH)], semw).wait()

    start(0, bufA)

    def body2(g, carry):
        @pl.when(g > 0)
        def _():
            drain_w(bufB)

        start(2 * g + 1, bufB)
        wait_g(bufA)
        compute(bufA)
        start_w(2 * g, bufA)
        wait_g(bufB)
        compute(bufB)
        start_w(2 * g + 1, bufB)
        drain_w(bufA)
        start(2 * g + 2, bufA)
        return carry

    lax.fori_loop(0, NCHUNK // 2, body2, 0)
    drain_w(bufB)
    wait_g(bufA)
    compute(bufA)
    start_w(NCHUNK - 1, bufA)
    drain_w(bufA)


def _silu(x):
    return x * jax.nn.sigmoid(x)


def _msg_body(gcol_ref, grow_ref, ef_ref, rdsq_ref,
              wtail_ref, w1l_ref, b1_ref, w2_ref, b2_ref,
              mr_ref, mc_ref):
    rd = rdsq_ref[...]
    et = (jnp.dot(ef_ref[...], wtail_ref[...],
                  preferred_element_type=jnp.float32)
          + rd * w1l_ref[...] + b1_ref[...])
    w2b = w2_ref[...].astype(jnp.bfloat16)
    hr = _silu(gcol_ref[...] + et).astype(jnp.bfloat16)
    mr_ref[...] = _silu(jnp.dot(hr, w2b,
                                preferred_element_type=jnp.float32)
                        + b2_ref[...])
    hc = _silu(grow_ref[...] + et).astype(jnp.bfloat16)
    mc_ref[...] = _silu(jnp.dot(hc, w2b,
                                preferred_element_type=jnp.float32)
                        + b2_ref[...])


def _tc_msg(gcol, grow, ef, rdsq, wtail, w1l, b1, w2, b2):
    full = lambda r, c: pl.BlockSpec((r, c), lambda i: (0, 0))
    return pl.pallas_call(
        _msg_body,
        grid=(E // BE,),
        in_specs=[
            pl.BlockSpec((BE, H), lambda i: (i, 0)),
            pl.BlockSpec((BE, H), lambda i: (i, 0)),
            pl.BlockSpec((BE, 16), lambda i: (i, 0)),
            pl.BlockSpec((BE, 1), lambda i: (i, 0)),
            full(16, H), full(1, H), full(1, H), full(H, H), full(1, H),
        ],
        out_specs=[
            pl.BlockSpec((BE, H), lambda i: (i, 0)),
            pl.BlockSpec((BE, H), lambda i: (i, 0)),
        ],
        out_shape=[
            jax.ShapeDtypeStruct((E, H), jnp.float32),
            jax.ShapeDtypeStruct((E, H), jnp.float32),
        ],
    )(gcol, grow, ef, rdsq, wtail, w1l, b1, w2, b2)


@functools.partial(
    pl.kernel,
    mesh=_sc_mesh,
    compiler_params=_sc_params,
    out_type=[
        jax.ShapeDtypeStruct((2 * N, H), jnp.float32),
    ],
    scratch_types=[
        pltpu.VMEM_SHARED((N, H), jnp.float32),
        pltpu.VMEM((ZR, H), jnp.float32),
        pltpu.VMEM((CH,), jnp.int32), pltpu.VMEM((CH,), jnp.int32),
        pltpu.VMEM((CH,), jnp.int32), pltpu.VMEM((CH,), jnp.int32),
        pltpu.VMEM((CH, H), jnp.float32), pltpu.VMEM((CH, H), jnp.float32),
        pltpu.VMEM((CH, H), jnp.float32), pltpu.VMEM((CH, H), jnp.float32),
        pltpu.SemaphoreType.DMA, pltpu.SemaphoreType.DMA,
    ],
)
def _sc_scatter(mr_hbm, mc_hbm, row_hbm, col_hbm,
                part_hbm,
                acc_sh, zbuf_v,
                ir0, ic0, ir1, ic1, mr0, mc0, mr1, mc1, sem0, sem1):
    cid = lax.axis_index("c")
    sid = lax.axis_index("s")
    base0 = (cid * NS + sid) * EPW

    def zb(r, carry):
        for k in range(H // 16):
            zbuf_v[r, pl.ds(k * 16, 16)] = jnp.zeros((16,), jnp.float32)
        return carry

    lax.fori_loop(0, ZR, zb, 0)

    def zc(q, carry):
        pltpu.sync_copy(zbuf_v, acc_sh.at[pl.ds(sid * RPT + q * ZR, ZR), :])
        return carry

    lax.fori_loop(0, NZQ, zc, 0)
    plsc.subcore_barrier()

    bufA = (ir0, ic0, mr0, mc0, sem0)
    bufB = (ir1, ic1, mr1, mc1, sem1)

    def start(i, buf):
        ir, ic, mrv, mcv, sem = buf
        b = base0 + i * CH
        pltpu.async_copy(row_hbm.at[pl.ds(b, CH)], ir, sem)
        pltpu.async_copy(col_hbm.at[pl.ds(b, CH)], ic, sem)
        pltpu.async_copy(mr_hbm.at[pl.ds(b, CH)], mrv, sem)
        pltpu.async_copy(mc_hbm.at[pl.ds(b, CH)], mcv, sem)

    def finish(buf):
        ir, ic, mrv, mcv, sem = buf
        pltpu.make_async_copy(row_hbm.at[pl.ds(0, CH)], ir, sem).wait()
        pltpu.make_async_copy(col_hbm.at[pl.ds(0, CH)], ic, sem).wait()
        pltpu.make_async_copy(mr_hbm.at[pl.ds(0, CH)], mrv, sem).wait()
        pltpu.make_async_copy(mc_hbm.at[pl.ds(0, CH)], mcv, sem).wait()
        pltpu.sync_copy(mrv, acc_sh.at[ir], add=True)
        pltpu.sync_copy(mcv, acc_sh.at[ic], add=True)

    start(0, bufA)

    def body2(g, carry):
        start(2 * g + 1, bufB)
        finish(bufA)
        start(2 * g + 2, bufA)
        finish(bufB)
        return carry

    lax.fori_loop(0, NCHUNK // 2, body2, 0)
    finish(bufA)
    plsc.subcore_barrier()

    for q in range(NQ):
        r0 = sid * RPT + q * RQ
        pltpu.sync_copy(acc_sh.at[pl.ds(r0, RQ), :],
                        part_hbm.at[pl.ds(cid * N + r0, RQ), :])


def _upd_body(nf_ref, p0_ref, p1_ref,
              u1a_ref, u1b_ref, ub1_ref, uw2_ref, ub2_ref, g_ref, bb_ref,
              out_ref):
    nf = nf_ref[...]
    agg = p0_ref[...] + p1_ref[...]
    h = _silu(jnp.dot(nf, u1a_ref[...], preferred_element_type=jnp.float32)
              + jnp.dot(agg, u1b_ref[...], preferred_element_type=jnp.float32)
              + ub1_ref[...])
    upd = jnp.dot(h, uw2_ref[...],
                  preferred_element_type=jnp.float32) + ub2_ref[...]
    x = nf + upd
    mu = jnp.mean(x, axis=-1, keepdims=True)
    var = jnp.mean((x - mu) ** 2, axis=-1, keepdims=True)
    out_ref[...] = (x - mu) * lax.rsqrt(var + 1e-5) * g_ref[...] + bb_ref[...]


def _tc_update(nf, p0, p1, u1a, u1b, ub1, uw2, ub2, g, bb):
    full = lambda r, c: pl.BlockSpec((r, c), lambda i: (0, 0))
    return pl.pallas_call(
        _upd_body,
        grid=(pl.cdiv(N, BN),),
        in_specs=[
            pl.BlockSpec((BN, D), lambda i: (i, 0)),
            pl.BlockSpec((BN, H), lambda i: (i, 0)),
            pl.BlockSpec((BN, H), lambda i: (i, 0)),
            full(D, H), full(H, H), full(1, H), full(H, D), full(1, D),
            full(1, D), full(1, D),
        ],
        out_specs=pl.BlockSpec((BN, D), lambda i: (i, 0)),
        out_shape=jax.ShapeDtypeStruct((N, D), jnp.float32),
    )(nf, p0, p1, u1a, u1b, ub1, uw2, ub2, g, bb)


def kernel(node_features, edge_features, positions,
           msg_w1, msg_b1, msg_w2, msg_b2,
           upd_w1, upd_b1, upd_w2, upd_b2,
           ln_g, ln_b, edge_index):
    row = edge_index[0].astype(jnp.int32)
    col = edge_index[1].astype(jnp.int32)

    w1_nf = msg_w1[:D]
    wtail = jnp.zeros((16, H), jnp.float32).at[3:16].set(msg_w1[D:D + 13])
    w1l = msg_w1[141:142]
    b1 = msg_b1[None]
    b2 = msg_b2[None]
    u1a = upd_w1[:D]
    u1b = upd_w1[D:]
    ub1 = upd_b1[None]
    ub2 = upd_b2[None]
    g = ln_g[None]
    bb = ln_b[None]

    pos16 = jnp.pad(positions, ((0, 0), (0, 13)))

    p = _tc_prep(node_features, w1_nf)
    gcol, grow, rdsq = _sc_gather(p, pos16, row, col)
    mr, mc = _tc_msg(gcol, grow, edge_features, rdsq.reshape(E, 1),
                     wtail, w1l, b1, msg_w2, b2)
    part, = _sc_scatter(mr, mc, row, col)
    return _tc_update(node_features, part[:N], part[N:],
                      u1a, u1b, ub1, upd_w2, ub2, g, bb)
